# baseline (speedup 1.0000x reference)
import functools

import jax
import jax.numpy as jnp
from jax import lax
from jax.experimental import pallas as pl
from jax.experimental.pallas import tpu as pltpu
from jax.experimental.pallas import tpu_sc as plsc

N = 10000
E = 320000
D = 128
K = 3
NC, NS = 2, 16
NT = NC * NS
PADN = 10240
EPT = E // NT
CH = 125
CPH = EPT // CH
NB = 40
NBA = 4
NBB = 2
RPT = PADN // NS
BR = 1280
GRID = PADN // BR

_mesh = plsc.VectorSubcoreMesh(
    core_axis_name="c", subcore_axis_name="s", num_cores=NC, num_subcores=NS
)


@functools.partial(
    pl.kernel,
    out_type=jax.ShapeDtypeStruct((NC * K * PADN,), jnp.float32),
    mesh=_mesh,
    scratch_types=[
        pltpu.VMEM((CPH, CH), jnp.int32),
        pltpu.VMEM((CH,), jnp.float32),
        pltpu.VMEM((RPT,), jnp.float32),
        pltpu.VMEM_SHARED((PADN,), jnp.float32),
        pltpu.VMEM_SHARED((PADN,), jnp.float32),
        pltpu.VMEM_SHARED((PADN,), jnp.float32),
        pltpu.SemaphoreType.DMA,
    ],
)
def _deg_kernel(dst_hbm, ones_hbm, z_hbm, out_hbm, didx, ones_v, z_v,
                d0, d1, d2, sem):
    c = lax.axis_index("c")
    s = lax.axis_index("s")
    pltpu.sync_copy(ones_hbm, ones_v)
    pltpu.sync_copy(z_hbm, z_v)
    degs = (d0, d1, d2)
    base = s * RPT
    for k in range(K):
        pltpu.sync_copy(z_v, degs[k].at[pl.ds(base, RPT)])
    plsc.subcore_barrier()
    for k in range(K):
        pltpu.sync_copy(dst_hbm.at[k, c, s], didx)

        def body(j, carry, _deg=degs[k]):
            pltpu.async_copy(ones_v, _deg.at[didx.at[j]], sem, add=True)
            return carry

        lax.fori_loop(0, CPH, body, 0)

        def drain(j, carry, _deg=degs[k]):
            pltpu.make_async_copy(ones_v, _deg.at[didx.at[0]], sem).wait()
            return carry

        lax.fori_loop(0, CPH, drain, 0)
    plsc.subcore_barrier()
    for k in range(K):
        pltpu.sync_copy(
            degs[k].at[pl.ds(base, RPT)],
            out_hbm.at[pl.ds((k * NC + c) * PADN + base, RPT)],
        )


@functools.partial(
    pl.kernel,
    out_type=jax.ShapeDtypeStruct((NC, 2, PADN, D), jnp.float32),
    mesh=_mesh,
    scratch_types=[
        pltpu.VMEM((NB, CH), jnp.int32),
        pltpu.VMEM((NB, CH), jnp.int32),
        pltpu.VMEM((CH, D), jnp.float32),
        pltpu.VMEM((CH, D), jnp.float32),
        pltpu.VMEM_SHARED((PADN, D), jnp.float32),
        pltpu.SemaphoreType.DMA,
        pltpu.SemaphoreType.DMA,
        pltpu.SemaphoreType.DMA,
        pltpu.SemaphoreType.DMA,
    ],
)
def _edge_kernel(gt, srcA, dstA, srcB, dstB, z_hbm, out_hbm,
                 sidx, didx, b0, b1, acc, gs0, gs1, ss0, ss1):
    c = lax.axis_index("c")
    s = lax.axis_index("s")
    base = s * RPT

    def g_start(j, buf, sem):
        pltpu.async_copy(gt.at[sidx.at[j]], buf, sem)

    def g_wait(buf, sem):
        pltpu.make_async_copy(gt.at[sidx.at[0]], buf, sem).wait()

    def s_start(j, buf, sem):
        pltpu.async_copy(buf, acc.at[didx.at[j]], sem, add=True)

    def s_wait(buf, sem):
        pltpu.make_async_copy(buf, acc.at[didx.at[0]], sem).wait()

    def zero_own_rows():
        pltpu.sync_copy(z_hbm, b0)
        for z in range(RPT // CH):
            pltpu.sync_copy(b0, acc.at[pl.ds(base + z * CH, CH)])
        ntail = RPT - (RPT // CH) * CH
        pltpu.sync_copy(
            b0.at[pl.ds(0, ntail)],
            acc.at[pl.ds(base + (RPT // CH) * CH, ntail)],
        )

    def run_batch(src_r, dst_r, h):
        pltpu.sync_copy(src_r.at[c, s, h], sidx)
        pltpu.sync_copy(dst_r.at[c, s, h], didx)
        g_start(0, b0, gs0)
        g_wait(b0, gs0)
        s_start(0, b0, ss0)
        g_start(1, b1, gs1)

        def body(m, carry):
            j1 = 2 * m + 1
            g_wait(b1, gs1)
            s_start(j1, b1, ss1)
            s_wait(b0, ss0)
            g_start(j1 + 1, b0, gs0)
            j2 = 2 * m + 2
            g_wait(b0, gs0)
            s_start(j2, b0, ss0)
            s_wait(b1, ss1)
            g_start(j2 + 1, b1, gs1)
            return carry

        lax.fori_loop(0, (NB - 2) // 2, body, 0)
        g_wait(b1, gs1)
        s_start(NB - 1, b1, ss1)
        s_wait(b0, ss0)
        s_wait(b1, ss1)

    zero_own_rows()
    plsc.subcore_barrier()
    for h in range(NBA):
        run_batch(srcA, dstA, h)
    plsc.subcore_barrier()
    pltpu.sync_copy(acc.at[pl.ds(base, RPT)], out_hbm.at[c, 0, pl.ds(base, RPT)])
    zero_own_rows()
    plsc.subcore_barrier()
    for h in range(NBB):
        run_batch(srcB, dstB, h)
    plsc.subcore_barrier()
    pltpu.sync_copy(acc.at[pl.ds(base, RPT)], out_hbm.at[c, 1, pl.ds(base, RPT)])


def _prep_body(x_ref, w_ref, degp_ref, g_ref, dinv_ref):
    degp = degp_ref[...]
    dinv = lax.rsqrt(degp[0, 0] + degp[0, 1] + 1.0)
    dinv_ref[0, 0] = dinv
    h = jnp.dot(x_ref[...], w_ref[0], preferred_element_type=jnp.float32)
    g_ref[...] = h * dinv[:, None]


_prep = pl.pallas_call(
    _prep_body,
    grid=(K, GRID),
    in_specs=[
        pl.BlockSpec((BR, D), lambda k, i: (i, 0)),
        pl.BlockSpec((1, D, D), lambda k, i: (k, 0, 0)),
        pl.BlockSpec((1, NC, BR), lambda k, i: (k, 0, i)),
    ],
    out_specs=[
        pl.BlockSpec((BR, D), lambda k, i: (k * GRID + i, 0)),
        pl.BlockSpec((1, 1, BR), lambda k, i: (k, 0, i)),
    ],
    out_shape=[
        jax.ShapeDtypeStruct((K * PADN, D), jnp.float32),
        jax.ShapeDtypeStruct((K, 1, PADN), jnp.float32),
    ],
)


def _final_body(accp_ref, g0, g1, g2, dinv_ref, out_ref):
    dinv = dinv_ref[...]
    a = accp_ref[...]
    total = dinv[0][:, None] * (a[0, 0] + g0[...])
    total = total + dinv[1][:, None] * (a[1, 0] + g1[...])
    total = total + dinv[2][:, None] * (a[0, 1] + a[1, 1] + g2[...])
    out_ref[...] = total


_final = pl.pallas_call(
    _final_body,
    grid=(GRID,),
    in_specs=[
        pl.BlockSpec((NC, 2, BR, D), lambda i: (0, 0, i, 0)),
        pl.BlockSpec((BR, D), lambda i: (i, 0)),
        pl.BlockSpec((BR, D), lambda i: (GRID + i, 0)),
        pl.BlockSpec((BR, D), lambda i: (2 * GRID + i, 0)),
        pl.BlockSpec((K, BR), lambda i: (0, i)),
    ],
    out_specs=pl.BlockSpec((BR, D), lambda i: (i, 0)),
    out_shape=jax.ShapeDtypeStruct((PADN, D), jnp.float32),
)


def kernel(x, adj0, adj1, adj2, W0, W1, W2):
    src = jnp.stack([adj0[0], adj1[0], adj2[0]]).astype(jnp.int32)
    dst = jnp.stack([adj0[1], adj1[1], adj2[1]]).astype(jnp.int32)
    dstr = dst.reshape(K, NC, NS, CPH, CH)
    srcA = (src[:NC] + (jnp.arange(NC, dtype=jnp.int32) * PADN)[:, None]
            ).reshape(NC, NS, NBA, NB, CH)
    dstA = dst[:NC].reshape(NC, NS, NBA, NB, CH)
    srcB = (src[2] + 2 * PADN).reshape(NC, NS, NBB, NB, CH)
    dstB = dst[2].reshape(NC, NS, NBB, NB, CH)
    xp = jnp.pad(x.astype(jnp.float32), ((0, PADN - N), (0, 0)))
    ones_ch = jnp.ones((CH,), jnp.float32)
    z_rpt = jnp.zeros((RPT,), jnp.float32)
    z_rows = jnp.zeros((CH, D), jnp.float32)
    degp = _deg_kernel(dstr, ones_ch, z_rpt).reshape(K, NC, PADN)
    wk = jnp.stack([W0, W1, W2])
    gt, dinv = _prep(xp, wk, degp)
    accp = _edge_kernel(gt, srcA, dstA, srcB, dstB, z_rows)
    out = _final(accp, gt, gt, gt, dinv.reshape(K, PADN))
    return out[:N]


# speedup vs baseline: 1.0350x; 1.0350x over previous
import functools

import jax
import jax.numpy as jnp
from jax import lax
from jax.experimental import pallas as pl
from jax.experimental.pallas import tpu as pltpu
from jax.experimental.pallas import tpu_sc as plsc

N = 10000
E = 320000
D = 128
K = 3
NC, NS = 2, 16
NT = NC * NS
PADN = 10240
EPT = E // NT
CH = 125
CPH = EPT // CH
NB = 40
NBA = 4
NBB = 2
RPT = PADN // NS
BR = 1280
GRID = PADN // BR

_mesh = plsc.VectorSubcoreMesh(
    core_axis_name="c", subcore_axis_name="s", num_cores=NC, num_subcores=NS
)


@functools.partial(
    pl.kernel,
    out_type=jax.ShapeDtypeStruct((NC * K * PADN,), jnp.float32),
    mesh=_mesh,
    scratch_types=[
        pltpu.VMEM((CPH, CH), jnp.int32),
        pltpu.VMEM((CH,), jnp.float32),
        pltpu.VMEM((RPT,), jnp.float32),
        pltpu.VMEM_SHARED((PADN,), jnp.float32),
        pltpu.VMEM_SHARED((PADN,), jnp.float32),
        pltpu.VMEM_SHARED((PADN,), jnp.float32),
        pltpu.SemaphoreType.DMA,
    ],
)
def _deg_kernel(dst_hbm, ones_hbm, z_hbm, out_hbm, didx, ones_v, z_v,
                d0, d1, d2, sem):
    c = lax.axis_index("c")
    s = lax.axis_index("s")
    pltpu.sync_copy(ones_hbm, ones_v)
    pltpu.sync_copy(z_hbm, z_v)
    degs = (d0, d1, d2)
    base = s * RPT
    for k in range(K):
        pltpu.sync_copy(z_v, degs[k].at[pl.ds(base, RPT)])
    plsc.subcore_barrier()
    for k in range(K):
        pltpu.sync_copy(dst_hbm.at[k, c, s], didx)

        def body(j, carry, _deg=degs[k]):
            pltpu.async_copy(ones_v, _deg.at[didx.at[j]], sem, add=True)
            return carry

        lax.fori_loop(0, CPH, body, 0)

        def drain(j, carry, _deg=degs[k]):
            pltpu.make_async_copy(ones_v, _deg.at[didx.at[0]], sem).wait()
            return carry

        lax.fori_loop(0, CPH, drain, 0)
    plsc.subcore_barrier()
    for k in range(K):
        pltpu.sync_copy(
            degs[k].at[pl.ds(base, RPT)],
            out_hbm.at[pl.ds((c * K + k) * PADN + base, RPT)],
        )


@functools.partial(
    pl.kernel,
    out_type=jax.ShapeDtypeStruct((NC, 2, PADN, D), jnp.float32),
    mesh=_mesh,
    scratch_types=[
        pltpu.VMEM((NB, CH), jnp.int32),
        pltpu.VMEM((NB, CH), jnp.int32),
        pltpu.VMEM((CH, D), jnp.float32),
        pltpu.VMEM((CH, D), jnp.float32),
        pltpu.VMEM_SHARED((PADN, D), jnp.float32),
        pltpu.SemaphoreType.DMA,
        pltpu.SemaphoreType.DMA,
        pltpu.SemaphoreType.DMA,
        pltpu.SemaphoreType.DMA,
    ],
)
def _edge_kernel(g0, g1, g2, srcA, dstA, srcB, dstB, z_hbm, out_hbm,
                 sidx, didx, b0, b1, acc, gs0, gs1, ss0, ss1):
    c = lax.axis_index("c")
    s = lax.axis_index("s")
    base = s * RPT

    def s_start(j, buf, sem):
        pltpu.async_copy(buf, acc.at[didx.at[j]], sem, add=True)

    def s_wait(buf, sem):
        pltpu.make_async_copy(buf, acc.at[didx.at[0]], sem).wait()

    def zero_own_rows():
        pltpu.sync_copy(z_hbm, b0)
        for z in range(RPT // CH):
            pltpu.sync_copy(b0, acc.at[pl.ds(base + z * CH, CH)])
        pltpu.sync_copy(
            b0.at[pl.ds(0, RPT - (RPT // CH) * CH)],
            acc.at[pl.ds(base + (RPT // CH) * CH, RPT - (RPT // CH) * CH)],
        )

    def run_batch(gk, src_r, dst_r, h):
        def g_start(j, buf, sem):
            pltpu.async_copy(gk.at[sidx.at[j]], buf, sem)

        def g_wait(buf, sem):
            pltpu.make_async_copy(gk.at[sidx.at[0]], buf, sem).wait()

        pltpu.sync_copy(src_r.at[c, s, h], sidx)
        pltpu.sync_copy(dst_r.at[c, s, h], didx)
        g_start(0, b0, gs0)
        g_wait(b0, gs0)
        s_start(0, b0, ss0)
        g_start(1, b1, gs1)

        def body(m, carry):
            j1 = 2 * m + 1
            g_wait(b1, gs1)
            s_start(j1, b1, ss1)
            s_wait(b0, ss0)
            g_start(j1 + 1, b0, gs0)
            j2 = 2 * m + 2
            g_wait(b0, gs0)
            s_start(j2, b0, ss0)
            s_wait(b1, ss1)
            g_start(j2 + 1, b1, gs1)
            return carry

        lax.fori_loop(0, (NB - 2) // 2, body, 0)
        g_wait(b1, gs1)
        s_start(NB - 1, b1, ss1)
        s_wait(b0, ss0)
        s_wait(b1, ss1)

    zero_own_rows()
    plsc.subcore_barrier()

    @pl.when(c == 0)
    def _():
        for h in range(NBA):
            run_batch(g0, srcA, dstA, h)

    @pl.when(c == 1)
    def _():
        for h in range(NBA):
            run_batch(g1, srcA, dstA, h)

    plsc.subcore_barrier()
    pltpu.sync_copy(acc.at[pl.ds(base, RPT)], out_hbm.at[c, 0, pl.ds(base, RPT)])
    zero_own_rows()
    plsc.subcore_barrier()
    for h in range(NBB):
        run_batch(g2, srcB, dstB, h)
    plsc.subcore_barrier()
    pltpu.sync_copy(acc.at[pl.ds(base, RPT)], out_hbm.at[c, 1, pl.ds(base, RPT)])


def _prep_body(x_ref, w0, w1, w2, degp_ref, g0, g1, g2, dinv_ref):
    degp = degp_ref[...]
    dinv = lax.rsqrt(degp[0] + degp[1] + 1.0)
    dinv_ref[...] = dinv
    for k, (wr, gr) in enumerate(((w0, g0), (w1, g1), (w2, g2))):
        h = jnp.dot(x_ref[...], wr[...], preferred_element_type=jnp.float32)
        gr[...] = h * dinv[k][:, None]


_prep = pl.pallas_call(
    _prep_body,
    grid=(GRID,),
    in_specs=[
        pl.BlockSpec((BR, D), lambda i: (i, 0)),
        pl.BlockSpec((D, D), lambda i: (0, 0)),
        pl.BlockSpec((D, D), lambda i: (0, 0)),
        pl.BlockSpec((D, D), lambda i: (0, 0)),
        pl.BlockSpec((NC, K, BR), lambda i: (0, 0, i)),
    ],
    out_specs=[
        pl.BlockSpec((BR, D), lambda i: (i, 0)),
        pl.BlockSpec((BR, D), lambda i: (i, 0)),
        pl.BlockSpec((BR, D), lambda i: (i, 0)),
        pl.BlockSpec((K, BR), lambda i: (0, i)),
    ],
    out_shape=[
        jax.ShapeDtypeStruct((PADN, D), jnp.float32),
        jax.ShapeDtypeStruct((PADN, D), jnp.float32),
        jax.ShapeDtypeStruct((PADN, D), jnp.float32),
        jax.ShapeDtypeStruct((K, PADN), jnp.float32),
    ],
)


def _final_body(accp_ref, g0, g1, g2, dinv_ref, out_ref):
    dinv = dinv_ref[...]
    a = accp_ref[...]
    total = dinv[0][:, None] * (a[0, 0] + g0[...])
    total = total + dinv[1][:, None] * (a[1, 0] + g1[...])
    total = total + dinv[2][:, None] * (a[0, 1] + a[1, 1] + g2[...])
    out_ref[...] = total


_final = pl.pallas_call(
    _final_body,
    grid=(GRID,),
    in_specs=[
        pl.BlockSpec((NC, 2, BR, D), lambda i: (0, 0, i, 0)),
        pl.BlockSpec((BR, D), lambda i: (i, 0)),
        pl.BlockSpec((BR, D), lambda i: (i, 0)),
        pl.BlockSpec((BR, D), lambda i: (i, 0)),
        pl.BlockSpec((K, BR), lambda i: (0, i)),
    ],
    out_specs=pl.BlockSpec((BR, D), lambda i: (i, 0)),
    out_shape=jax.ShapeDtypeStruct((PADN, D), jnp.float32),
)


def kernel(x, adj0, adj1, adj2, W0, W1, W2):
    src = jnp.stack([adj0[0], adj1[0], adj2[0]]).astype(jnp.int32)
    dst = jnp.stack([adj0[1], adj1[1], adj2[1]]).astype(jnp.int32)
    dstr = dst.reshape(K, NC, NS, CPH, CH)
    srcA = src[:NC].reshape(NC, NS, NBA, NB, CH)
    dstA = dst[:NC].reshape(NC, NS, NBA, NB, CH)
    srcB = src[2].reshape(NC, NS, NBB, NB, CH)
    dstB = dst[2].reshape(NC, NS, NBB, NB, CH)
    xp = jnp.pad(x.astype(jnp.float32), ((0, PADN - N), (0, 0)))
    ones_ch = jnp.ones((CH,), jnp.float32)
    z_rpt = jnp.zeros((RPT,), jnp.float32)
    z_rows = jnp.zeros((CH, D), jnp.float32)
    degp = _deg_kernel(dstr, ones_ch, z_rpt).reshape(NC, K, PADN)
    g0, g1, g2, dinv = _prep(xp, W0, W1, W2, degp)
    accp = _edge_kernel(g0, g1, g2, srcA, dstA, srcB, dstB, z_rows)
    out = _final(accp, g0, g1, g2, dinv)
    return out[:N]


# speedup vs baseline: 1.0576x; 1.0219x over previous
import functools

import jax
import jax.numpy as jnp
from jax import lax
from jax.experimental import pallas as pl
from jax.experimental.pallas import tpu as pltpu
from jax.experimental.pallas import tpu_sc as plsc

N = 10000
E = 320000
D = 128
K = 3
NC, NS = 2, 16
NT = NC * NS
PADN = 10240
EPT = E // NT
CH = 125
CPH = EPT // CH
NB = 40
RPT = PADN // NS
BR = 1280
GRID = PADN // BR

_mesh = plsc.VectorSubcoreMesh(
    core_axis_name="c", subcore_axis_name="s", num_cores=NC, num_subcores=NS
)


@functools.partial(
    pl.kernel,
    out_type=jax.ShapeDtypeStruct((NC * K * PADN,), jnp.float32),
    mesh=_mesh,
    scratch_types=[
        pltpu.VMEM((CPH, CH), jnp.int32),
        pltpu.VMEM((CH,), jnp.float32),
        pltpu.VMEM((RPT,), jnp.float32),
        pltpu.VMEM_SHARED((PADN,), jnp.float32),
        pltpu.VMEM_SHARED((PADN,), jnp.float32),
        pltpu.VMEM_SHARED((PADN,), jnp.float32),
        pltpu.SemaphoreType.DMA,
    ],
)
def _deg_kernel(dst_hbm, ones_hbm, z_hbm, out_hbm, didx, ones_v, z_v,
                d0, d1, d2, sem):
    c = lax.axis_index("c")
    s = lax.axis_index("s")
    pltpu.sync_copy(ones_hbm, ones_v)
    pltpu.sync_copy(z_hbm, z_v)
    degs = (d0, d1, d2)
    base = s * RPT
    for k in range(K):
        pltpu.sync_copy(z_v, degs[k].at[pl.ds(base, RPT)])
    plsc.subcore_barrier()
    for k in range(K):
        pltpu.sync_copy(dst_hbm.at[k, c, s], didx)

        def body(j, carry, _deg=degs[k]):
            pltpu.async_copy(ones_v, _deg.at[didx.at[j]], sem, add=True)
            return carry

        lax.fori_loop(0, CPH, body, 0)

        def drain(j, carry, _deg=degs[k]):
            pltpu.make_async_copy(ones_v, _deg.at[didx.at[0]], sem).wait()
            return carry

        lax.fori_loop(0, CPH, drain, 0)
    plsc.subcore_barrier()
    for k in range(K):
        pltpu.sync_copy(
            degs[k].at[pl.ds(base, RPT)],
            out_hbm.at[pl.ds((c * K + k) * PADN + base, RPT)],
        )


@functools.partial(
    pl.kernel,
    out_type=jax.ShapeDtypeStruct((NC, K, PADN, D), jnp.float32),
    mesh=_mesh,
    scratch_types=[
        pltpu.VMEM((NB, CH), jnp.int32),
        pltpu.VMEM((NB, CH), jnp.int32),
        pltpu.VMEM((CH, D), jnp.float32),
        pltpu.VMEM((CH, D), jnp.float32),
        pltpu.VMEM_SHARED((PADN, D), jnp.float32),
        pltpu.SemaphoreType.DMA,
        pltpu.SemaphoreType.DMA,
        pltpu.SemaphoreType.DMA,
        pltpu.SemaphoreType.DMA,
    ],
)
def _edge_kernel(g0, g1, g2, src_hbm, dst_hbm, z_hbm, out_hbm,
                 sidx, didx, b0, b1, acc, gs0, gs1, ss0, ss1):
    c = lax.axis_index("c")
    s = lax.axis_index("s")
    gs = (g0, g1, g2)
    base = s * RPT

    for k in range(K):
        gk = gs[k]

        def g_start(j, buf, sem):
            pltpu.async_copy(gk.at[sidx.at[j]], buf, sem)

        def g_wait(buf, sem):
            pltpu.make_async_copy(gk.at[sidx.at[0]], buf, sem).wait()

        def s_start(j, buf, sem):
            pltpu.async_copy(buf, acc.at[didx.at[j]], sem, add=True)

        def s_wait(buf, sem):
            pltpu.make_async_copy(buf, acc.at[didx.at[0]], sem).wait()

        pltpu.sync_copy(z_hbm, b0)
        for z in range(RPT // CH):
            pltpu.sync_copy(b0, acc.at[pl.ds(base + z * CH, CH)])
        pltpu.sync_copy(
            b0.at[pl.ds(0, RPT - (RPT // CH) * CH)],
            acc.at[pl.ds(base + (RPT // CH) * CH, RPT - (RPT // CH) * CH)],
        )
        plsc.subcore_barrier()

        for h in range(CPH // NB):
            pltpu.sync_copy(src_hbm.at[k, c, s, pl.ds(h * NB, NB)], sidx)
            pltpu.sync_copy(dst_hbm.at[k, c, s, pl.ds(h * NB, NB)], didx)
            g_start(0, b0, gs0)
            g_wait(b0, gs0)
            s_start(0, b0, ss0)
            g_start(1, b1, gs1)

            def body(m, carry):
                j1 = 2 * m + 1
                g_wait(b1, gs1)
                s_start(j1, b1, ss1)
                s_wait(b0, ss0)
                g_start(j1 + 1, b0, gs0)
                j2 = 2 * m + 2
                g_wait(b0, gs0)
                s_start(j2, b0, ss0)
                s_wait(b1, ss1)
                g_start(j2 + 1, b1, gs1)
                return carry

            lax.fori_loop(0, (NB - 2) // 2, body, 0)
            g_wait(b1, gs1)
            s_start(NB - 1, b1, ss1)
            s_wait(b0, ss0)
            s_wait(b1, ss1)

        plsc.subcore_barrier()
        pltpu.sync_copy(
            acc.at[pl.ds(base, RPT)], out_hbm.at[c, k, pl.ds(base, RPT)]
        )
        plsc.subcore_barrier()


def _prep_body(x_ref, w0, w1, w2, degp_ref, g0, g1, g2, dinv_ref):
    degp = degp_ref[...]
    dinv = lax.rsqrt(degp[0] + degp[1] + 1.0)
    dinv_ref[...] = dinv
    for k, (wr, gr) in enumerate(((w0, g0), (w1, g1), (w2, g2))):
        h = jnp.dot(x_ref[...], wr[...], preferred_element_type=jnp.float32)
        gr[...] = h * dinv[k][:, None]


_prep = pl.pallas_call(
    _prep_body,
    grid=(GRID,),
    in_specs=[
        pl.BlockSpec((BR, D), lambda i: (i, 0)),
        pl.BlockSpec((D, D), lambda i: (0, 0)),
        pl.BlockSpec((D, D), lambda i: (0, 0)),
        pl.BlockSpec((D, D), lambda i: (0, 0)),
        pl.BlockSpec((NC, K, BR), lambda i: (0, 0, i)),
    ],
    out_specs=[
        pl.BlockSpec((BR, D), lambda i: (i, 0)),
        pl.BlockSpec((BR, D), lambda i: (i, 0)),
        pl.BlockSpec((BR, D), lambda i: (i, 0)),
        pl.BlockSpec((K, BR), lambda i: (0, i)),
    ],
    out_shape=[
        jax.ShapeDtypeStruct((PADN, D), jnp.float32),
        jax.ShapeDtypeStruct((PADN, D), jnp.float32),
        jax.ShapeDtypeStruct((PADN, D), jnp.float32),
        jax.ShapeDtypeStruct((K, PADN), jnp.float32),
    ],
)


def _final_body(accp_ref, g0, g1, g2, dinv_ref, out_ref):
    dinv = dinv_ref[...]
    acc = accp_ref[...]
    total = jnp.zeros(out_ref.shape, jnp.float32)
    for k, gr in enumerate((g0, g1, g2)):
        total = total + dinv[k][:, None] * (acc[0, k] + acc[1, k] + gr[...])
    out_ref[...] = total


_final = pl.pallas_call(
    _final_body,
    grid=(GRID,),
    in_specs=[
        pl.BlockSpec((NC, K, BR, D), lambda i: (0, 0, i, 0)),
        pl.BlockSpec((BR, D), lambda i: (i, 0)),
        pl.BlockSpec((BR, D), lambda i: (i, 0)),
        pl.BlockSpec((BR, D), lambda i: (i, 0)),
        pl.BlockSpec((K, BR), lambda i: (0, i)),
    ],
    out_specs=pl.BlockSpec((BR, D), lambda i: (i, 0)),
    out_shape=jax.ShapeDtypeStruct((PADN, D), jnp.float32),
)


def kernel(x, adj0, adj1, adj2, W0, W1, W2):
    src = jnp.stack([adj0[0], adj1[0], adj2[0]]).astype(jnp.int32)
    dst = jnp.stack([adj0[1], adj1[1], adj2[1]]).astype(jnp.int32)
    srcr = src.reshape(K, NC, NS, CPH, CH)
    dstr = dst.reshape(K, NC, NS, CPH, CH)
    xp = jnp.pad(x.astype(jnp.float32), ((0, PADN - N), (0, 0)))
    ones_ch = jnp.ones((CH,), jnp.float32)
    z_rpt = jnp.zeros((RPT,), jnp.float32)
    z_rows = jnp.zeros((CH, D), jnp.float32)
    degp = _deg_kernel(dstr, ones_ch, z_rpt).reshape(NC, K, PADN)
    g0, g1, g2, dinv = _prep(xp, W0, W1, W2, degp)
    accp = _edge_kernel(g0, g1, g2, srcr, dstr, z_rows)
    out = _final(accp, g0, g1, g2, dinv)
    return out[:N]
